# baseline (speedup 1.0000x reference)
import functools

import jax
import jax.numpy as jnp
from jax import lax
from jax.experimental import pallas as pl
from jax.experimental.pallas import tpu as pltpu
from jax.experimental.pallas import tpu_sc as plsc

N_NODES = 10000
N_EDGES = 320000
D = 128
NUM_CLASSES = 10

NC = 2
NS = 16
NW = NC * NS
EB = 50
NB = N_EDGES // (NW * EB)
NR = 4
CH = 8
BODY = 2 * CH
STEADY = NB - CH

CHUNK = 640
LAST_CHUNK = N_NODES - (NS - 1) * CHUNK
ZROWS = 40


def _sc_body(x_hbm, idx_hbm, out_hbm, idxs, rows, zbuf, agg_sh,
             semI, semG, semS):
    cid = lax.axis_index("c")
    sid = lax.axis_index("s")
    wid = cid * NS + sid
    srcA, dstA, srcB, dstB = idxs
    semSA, semDA, semSB, semDB = semI

    zrow = jnp.zeros((16,), jnp.float32)

    def zstore(r):
        for c in range(D // 16):
            zbuf[r, pl.ds(c * 16, 16)] = zrow

    pl.loop(0, ZROWS)(zstore)
    nz = lax.select(sid == NS - 1, LAST_CHUNK // ZROWS, CHUNK // ZROWS)

    def zchunk(k):
        pltpu.sync_copy(zbuf, agg_sh.at[pl.ds(sid * CHUNK + k * ZROWS, ZROWS)])

    pl.loop(0, nz)(zchunk)
    plsc.subcore_barrier()

    def g_wait(src_c, k, rs):
        pltpu.make_async_copy(x_hbm.at[src_c.at[k]], rows[rs], semG[rs]).wait()

    def g_start(src_c, k, rs):
        pltpu.async_copy(x_hbm.at[src_c.at[k]], rows[rs], semG[rs])

    def s_start(dst_c, k, rs):
        pltpu.async_copy(rows[rs], agg_sh.at[dst_c.at[k]], semS[rs], add=True)

    def s_wait(dst_c, k, rs):
        pltpu.make_async_copy(rows[rs], agg_sh.at[dst_c.at[k]], semS[rs]).wait()

    def chunk_start(src_c, dst_c, sem_s, sem_d, j):
        pltpu.async_copy(idx_hbm.at[0, wid, pl.ds(j, CH)], src_c, sem_s)
        pltpu.async_copy(idx_hbm.at[1, wid, pl.ds(j, CH)], dst_c, sem_d)

    def chunk_wait(src_c, dst_c, sem_s, sem_d, j):
        pltpu.make_async_copy(idx_hbm.at[0, wid, pl.ds(j, CH)], src_c,
                              sem_s).wait()
        pltpu.make_async_copy(idx_hbm.at[1, wid, pl.ds(j, CH)], dst_c,
                              sem_d).wait()

    pltpu.sync_copy(idx_hbm.at[0, wid, pl.ds(0, CH)], srcA)
    pltpu.sync_copy(idx_hbm.at[1, wid, pl.ds(0, CH)], dstA)
    pltpu.sync_copy(idx_hbm.at[0, wid, pl.ds(CH, CH)], srcB)
    pltpu.sync_copy(idx_hbm.at[1, wid, pl.ds(CH, CH)], dstB)
    for b in range(NR - 1):
        g_start(srcA, b, b)

    def turn(j, b, last_block):
        src_c, dst_c = (srcA, dstA) if b < CH else (srcB, dstB)
        k, rs = b % CH, b % NR
        pb = (b - 1) % BODY
        pdst = dstA if pb < CH else dstB
        g_wait(src_c, k, rs)
        s_start(dst_c, k, rs)
        if b == 0:
            @pl.when(j >= 1)
            def _drain0():
                s_wait(pdst, pb % CH, pb % NR)
        else:
            s_wait(pdst, pb % CH, pb % NR)
        if not last_block:
            if b == 1:
                @pl.when(j >= 1)
                def _refillB():
                    chunk_start(srcB, dstB, semSB, semDB, j + CH)
            if b == CH:
                chunk_start(srcA, dstA, semSA, semDA, j + BODY)
            if b == 5:
                @pl.when(j >= 1)
                def _waitB():
                    chunk_wait(srcB, dstB, semSB, semDB, j + CH)
            if b == CH + 5:
                chunk_wait(srcA, dstA, semSA, semDA, j + BODY)
            nb = b + NR - 1
            nsrc = srcA if (nb < CH or nb >= BODY) else srcB
            g_start(nsrc, nb % CH, nb % NR)
        else:
            if b + NR - 1 < CH:
                g_start(srcA, b + NR - 1, (b + NR - 1) % NR)

    def body(j):
        for b in range(BODY):
            turn(j, b, last_block=False)

    pl.loop(0, STEADY, step=BODY)(body)
    for b in range(CH):
        turn(STEADY, b, last_block=True)
    s_wait(dstA, CH - 1, (NB - 1) % NR)

    plsc.subcore_barrier()

    @pl.when(sid < NS - 1)
    def _w0():
        pltpu.sync_copy(agg_sh.at[pl.ds(sid * CHUNK, CHUNK)],
                        out_hbm.at[cid, pl.ds(sid * CHUNK, CHUNK)])

    @pl.when(sid == NS - 1)
    def _w1():
        pltpu.sync_copy(agg_sh.at[pl.ds((NS - 1) * CHUNK, LAST_CHUNK)],
                        out_hbm.at[cid, pl.ds((NS - 1) * CHUNK, LAST_CHUNK)])


@functools.partial(
    pl.kernel,
    out_type=jax.ShapeDtypeStruct((NC, N_NODES, D), jnp.float32),
    mesh=plsc.VectorSubcoreMesh(core_axis_name="c", subcore_axis_name="s",
                                num_cores=NC, num_subcores=NS),
    scratch_types=[
        tuple(pltpu.VMEM((CH, EB), jnp.int32) for _ in range(4)),
        tuple(pltpu.VMEM((EB, D), jnp.float32) for _ in range(NR)),
        pltpu.VMEM((ZROWS, D), jnp.float32),
        pltpu.VMEM_SHARED((N_NODES, D), jnp.float32),
        tuple(pltpu.SemaphoreType.DMA for _ in range(4)),
        tuple(pltpu.SemaphoreType.DMA for _ in range(NR)),
        tuple(pltpu.SemaphoreType.DMA for _ in range(NR)),
    ],
)
def _sc_aggregate(x_hbm, idx_hbm, out_hbm, idxs, rows, zbuf, agg_sh,
                  semI, semG, semS):
    _sc_body(x_hbm, idx_hbm, out_hbm, idxs, rows, zbuf, agg_sh,
             semI, semG, semS)


ROW_BLK = 1000
GRID = N_NODES // ROW_BLK


def _tc_body(x_ref, p_ref, ws_ref, wn_ref, bg_ref, w1_ref, b1_ref,
             w2_ref, b2_ref, out_ref, acc_ref):
    i = pl.program_id(0)

    @pl.when(i == 0)
    def _init():
        acc_ref[...] = jnp.zeros_like(acc_ref)

    xb = x_ref[...]
    ab = p_ref[0] + p_ref[1]
    h = (jnp.dot(xb, ws_ref[...], preferred_element_type=jnp.float32)
         + jnp.dot(ab, wn_ref[...], preferred_element_type=jnp.float32)
         + bg_ref[...])
    h = jnp.maximum(h, 0.0)
    acc_ref[...] += jnp.sum(h, axis=0, keepdims=True)

    @pl.when(i == GRID - 1)
    def _final():
        emb = acc_ref[...] * (1.0 / N_NODES)
        z = jnp.maximum(
            jnp.dot(emb, w1_ref[...], preferred_element_type=jnp.float32)
            + b1_ref[...], 0.0)
        out_ref[...] = (jnp.dot(z, w2_ref[...], preferred_element_type=jnp.float32)
                        + b2_ref[...])


def _tc_finish(x, partials, W_self, W_nbr, b_gnn, W1, b1, W2, b2):
    return pl.pallas_call(
        _tc_body,
        grid=(GRID,),
        in_specs=[
            pl.BlockSpec((ROW_BLK, D), lambda i: (i, 0)),
            pl.BlockSpec((NC, ROW_BLK, D), lambda i: (0, i, 0)),
            pl.BlockSpec((D, D), lambda i: (0, 0)),
            pl.BlockSpec((D, D), lambda i: (0, 0)),
            pl.BlockSpec((1, D), lambda i: (0, 0)),
            pl.BlockSpec((D, D), lambda i: (0, 0)),
            pl.BlockSpec((1, D), lambda i: (0, 0)),
            pl.BlockSpec((D, NUM_CLASSES), lambda i: (0, 0)),
            pl.BlockSpec((1, NUM_CLASSES), lambda i: (0, 0)),
        ],
        out_specs=pl.BlockSpec((1, NUM_CLASSES), lambda i: (0, 0)),
        out_shape=jax.ShapeDtypeStruct((1, NUM_CLASSES), jnp.float32),
        scratch_shapes=[pltpu.VMEM((1, D), jnp.float32)],
    )(x, partials, W_self, W_nbr, b_gnn, W1, b1, W2, b2)


RELAY_GRID = 4


def _relayout_body(in_ref, out_ref):
    out_ref[...] = in_ref[...].reshape(2, NW // RELAY_GRID, NB, EB)


def _relayout(ei):
    return pl.pallas_call(
        _relayout_body,
        grid=(RELAY_GRID,),
        in_specs=[pl.BlockSpec((2, N_EDGES // RELAY_GRID), lambda g: (0, g))],
        out_specs=pl.BlockSpec((2, NW // RELAY_GRID, NB, EB),
                               lambda g: (0, g, 0, 0)),
        out_shape=jax.ShapeDtypeStruct((2, NW, NB, EB), jnp.int32),
    )(ei)


def kernel(x, edge_index, W_self, W_nbr, b_gnn, W1, b1, W2, b2):
    idx = _relayout(edge_index.astype(jnp.int32))
    partials = _sc_aggregate(x, idx)
    return _tc_finish(x, partials,
                      W_self, W_nbr, b_gnn.reshape(1, D),
                      W1, b1.reshape(1, D), W2, b2.reshape(1, NUM_CLASSES))


# speedup vs baseline: 1.0206x; 1.0206x over previous
import functools

import jax
import jax.numpy as jnp
from jax import lax
from jax.experimental import pallas as pl
from jax.experimental.pallas import tpu as pltpu
from jax.experimental.pallas import tpu_sc as plsc

N_NODES = 10000
N_EDGES = 320000
D = 128
NUM_CLASSES = 10

NC = 2
NS = 16
NW = NC * NS
EB = 50
NB = N_EDGES // (NW * EB)
NR = 4
CH = 8
BODY = 2 * CH
STEADY = NB - CH

CHUNK = 640
LAST_CHUNK = N_NODES - (NS - 1) * CHUNK
ZROWS = 40


def _sc_body(x_hbm, idx_hbm, out_hbm, idxs, rows, zbuf, agg_sh,
             semI, semG, semS):
    cid = lax.axis_index("c")
    sid = lax.axis_index("s")
    wid = cid * NS + sid
    srcA, dstA, srcB, dstB = idxs
    semSA, semDA, semSB, semDB = semI

    zrow = jnp.zeros((16,), jnp.float32)

    def zstore(r):
        for c in range(D // 16):
            zbuf[r, pl.ds(c * 16, 16)] = zrow

    pl.loop(0, ZROWS)(zstore)
    nz = lax.select(sid == NS - 1, LAST_CHUNK // ZROWS, CHUNK // ZROWS)

    def zchunk(k):
        pltpu.sync_copy(zbuf, agg_sh.at[pl.ds(sid * CHUNK + k * ZROWS, ZROWS)])

    pl.loop(0, nz)(zchunk)
    plsc.subcore_barrier()

    def g_wait(src_c, k, rs):
        pltpu.make_async_copy(x_hbm.at[src_c.at[k]], rows[rs], semG[rs]).wait()

    def g_start(src_c, k, rs):
        pltpu.async_copy(x_hbm.at[src_c.at[k]], rows[rs], semG[rs])

    def s_start(dst_c, k, rs):
        pltpu.async_copy(rows[rs], agg_sh.at[dst_c.at[k]], semS[rs], add=True)

    def s_wait(dst_c, k, rs):
        pltpu.make_async_copy(rows[rs], agg_sh.at[dst_c.at[k]], semS[rs]).wait()

    def chunk_start(src_c, dst_c, sem_s, sem_d, j):
        pltpu.async_copy(idx_hbm.at[0, wid, pl.ds(j, CH)], src_c, sem_s)
        pltpu.async_copy(idx_hbm.at[1, wid, pl.ds(j, CH)], dst_c, sem_d)

    def chunk_wait(src_c, dst_c, sem_s, sem_d, j):
        pltpu.make_async_copy(idx_hbm.at[0, wid, pl.ds(j, CH)], src_c,
                              sem_s).wait()
        pltpu.make_async_copy(idx_hbm.at[1, wid, pl.ds(j, CH)], dst_c,
                              sem_d).wait()

    pltpu.sync_copy(idx_hbm.at[0, wid, pl.ds(0, CH)], srcA)
    pltpu.sync_copy(idx_hbm.at[1, wid, pl.ds(0, CH)], dstA)
    pltpu.sync_copy(idx_hbm.at[0, wid, pl.ds(CH, CH)], srcB)
    pltpu.sync_copy(idx_hbm.at[1, wid, pl.ds(CH, CH)], dstB)
    for b in range(NR - 1):
        g_start(srcA, b, b)

    def turn(j, b, last_block):
        src_c, dst_c = (srcA, dstA) if b < CH else (srcB, dstB)
        k, rs = b % CH, b % NR
        pb = (b - 1) % BODY
        pdst = dstA if pb < CH else dstB
        g_wait(src_c, k, rs)
        s_start(dst_c, k, rs)
        if b == 0:
            @pl.when(j >= 1)
            def _drain0():
                s_wait(pdst, pb % CH, pb % NR)
        else:
            s_wait(pdst, pb % CH, pb % NR)
        if not last_block:
            if b == 1:
                @pl.when(j >= 1)
                def _refillB():
                    chunk_start(srcB, dstB, semSB, semDB, j + CH)
            if b == CH:
                chunk_start(srcA, dstA, semSA, semDA, j + BODY)
            if b == 5:
                @pl.when(j >= 1)
                def _waitB():
                    chunk_wait(srcB, dstB, semSB, semDB, j + CH)
            if b == CH + 5:
                chunk_wait(srcA, dstA, semSA, semDA, j + BODY)
            nb = b + NR - 1
            nsrc = srcA if (nb < CH or nb >= BODY) else srcB
            g_start(nsrc, nb % CH, nb % NR)
        else:
            if b + NR - 1 < CH:
                g_start(srcA, b + NR - 1, (b + NR - 1) % NR)

    def body(j):
        for b in range(BODY):
            turn(j, b, last_block=False)

    pl.loop(0, STEADY, step=BODY)(body)
    for b in range(CH):
        turn(STEADY, b, last_block=True)
    s_wait(dstA, CH - 1, (NB - 1) % NR)

    plsc.subcore_barrier()

    @pl.when(sid < NS - 1)
    def _w0():
        pltpu.sync_copy(agg_sh.at[pl.ds(sid * CHUNK, CHUNK)],
                        out_hbm.at[cid, pl.ds(sid * CHUNK, CHUNK)])

    @pl.when(sid == NS - 1)
    def _w1():
        pltpu.sync_copy(agg_sh.at[pl.ds((NS - 1) * CHUNK, LAST_CHUNK)],
                        out_hbm.at[cid, pl.ds((NS - 1) * CHUNK, LAST_CHUNK)])


@functools.partial(
    pl.kernel,
    out_type=jax.ShapeDtypeStruct((NC, N_NODES, D), jnp.float32),
    mesh=plsc.VectorSubcoreMesh(core_axis_name="c", subcore_axis_name="s",
                                num_cores=NC, num_subcores=NS),
    scratch_types=[
        tuple(pltpu.VMEM((CH, EB), jnp.int32) for _ in range(4)),
        tuple(pltpu.VMEM((EB, D), jnp.float32) for _ in range(NR)),
        pltpu.VMEM((ZROWS, D), jnp.float32),
        pltpu.VMEM_SHARED((N_NODES, D), jnp.float32),
        tuple(pltpu.SemaphoreType.DMA for _ in range(4)),
        tuple(pltpu.SemaphoreType.DMA for _ in range(NR)),
        tuple(pltpu.SemaphoreType.DMA for _ in range(NR)),
    ],
)
def _sc_aggregate(x_hbm, idx_hbm, out_hbm, idxs, rows, zbuf, agg_sh,
                  semI, semG, semS):
    _sc_body(x_hbm, idx_hbm, out_hbm, idxs, rows, zbuf, agg_sh,
             semI, semG, semS)


ROW_BLK = 2000
GRID = N_NODES // ROW_BLK


def _tc_body(x_ref, p_ref, ws_ref, wn_ref, bg_ref, w1_ref, b1_ref,
             w2_ref, b2_ref, out_ref, acc_ref):
    i = pl.program_id(0)

    @pl.when(i == 0)
    def _init():
        acc_ref[...] = jnp.zeros_like(acc_ref)

    xb = x_ref[...]
    ab = p_ref[0] + p_ref[1]
    h = (jnp.dot(xb, ws_ref[...], preferred_element_type=jnp.float32)
         + jnp.dot(ab, wn_ref[...], preferred_element_type=jnp.float32)
         + bg_ref[...])
    h = jnp.maximum(h, 0.0)
    acc_ref[...] += jnp.sum(h, axis=0, keepdims=True)

    @pl.when(i == GRID - 1)
    def _final():
        emb = acc_ref[...] * (1.0 / N_NODES)
        z = jnp.maximum(
            jnp.dot(emb, w1_ref[...], preferred_element_type=jnp.float32)
            + b1_ref[...], 0.0)
        out_ref[...] = (jnp.dot(z, w2_ref[...], preferred_element_type=jnp.float32)
                        + b2_ref[...])


def _tc_finish(x, partials, W_self, W_nbr, b_gnn, W1, b1, W2, b2):
    return pl.pallas_call(
        _tc_body,
        grid=(GRID,),
        in_specs=[
            pl.BlockSpec((ROW_BLK, D), lambda i: (i, 0)),
            pl.BlockSpec((NC, ROW_BLK, D), lambda i: (0, i, 0)),
            pl.BlockSpec((D, D), lambda i: (0, 0)),
            pl.BlockSpec((D, D), lambda i: (0, 0)),
            pl.BlockSpec((1, D), lambda i: (0, 0)),
            pl.BlockSpec((D, D), lambda i: (0, 0)),
            pl.BlockSpec((1, D), lambda i: (0, 0)),
            pl.BlockSpec((D, NUM_CLASSES), lambda i: (0, 0)),
            pl.BlockSpec((1, NUM_CLASSES), lambda i: (0, 0)),
        ],
        out_specs=pl.BlockSpec((1, NUM_CLASSES), lambda i: (0, 0)),
        out_shape=jax.ShapeDtypeStruct((1, NUM_CLASSES), jnp.float32),
        scratch_shapes=[pltpu.VMEM((1, D), jnp.float32)],
    )(x, partials, W_self, W_nbr, b_gnn, W1, b1, W2, b2)


RELAY_GRID = 4


def _relayout_body(in_ref, out_ref):
    out_ref[...] = in_ref[...].reshape(2, NW // RELAY_GRID, NB, EB)


def _relayout(ei):
    return pl.pallas_call(
        _relayout_body,
        grid=(RELAY_GRID,),
        in_specs=[pl.BlockSpec((2, N_EDGES // RELAY_GRID), lambda g: (0, g))],
        out_specs=pl.BlockSpec((2, NW // RELAY_GRID, NB, EB),
                               lambda g: (0, g, 0, 0)),
        out_shape=jax.ShapeDtypeStruct((2, NW, NB, EB), jnp.int32),
    )(ei)


def kernel(x, edge_index, W_self, W_nbr, b_gnn, W1, b1, W2, b2):
    idx = _relayout(edge_index.astype(jnp.int32))
    partials = _sc_aggregate(x, idx)
    return _tc_finish(x, partials,
                      W_self, W_nbr, b_gnn.reshape(1, D),
                      W1, b1.reshape(1, D), W2, b2.reshape(1, NUM_CLASSES))


# speedup vs baseline: 1.0256x; 1.0049x over previous
import functools

import jax
import jax.numpy as jnp
from jax import lax
from jax.experimental import pallas as pl
from jax.experimental.pallas import tpu as pltpu
from jax.experimental.pallas import tpu_sc as plsc

N_NODES = 10000
N_EDGES = 320000
D = 128
NUM_CLASSES = 10

NC = 2
NS = 16
NW = NC * NS
EB = 50
NB = N_EDGES // (NW * EB)
NR = 4
CH = 8
BODY = 2 * CH
STEADY = NB - CH

CHUNK = 640
LAST_CHUNK = N_NODES - (NS - 1) * CHUNK
ZROWS = 40


def _sc_body(x_hbm, idx_hbm, out_hbm, idxs, rows, zbuf, agg_sh,
             semI, semG, semS):
    cid = lax.axis_index("c")
    sid = lax.axis_index("s")
    wid = cid * NS + sid
    srcA, dstA, srcB, dstB = idxs
    semSA, semDA, semSB, semDB = semI

    zrow = jnp.zeros((16,), jnp.float32)

    def zstore(r):
        for c in range(D // 16):
            zbuf[r, pl.ds(c * 16, 16)] = zrow

    pl.loop(0, ZROWS)(zstore)
    nz = lax.select(sid == NS - 1, LAST_CHUNK // ZROWS, CHUNK // ZROWS)

    def zchunk(k):
        pltpu.sync_copy(zbuf, agg_sh.at[pl.ds(sid * CHUNK + k * ZROWS, ZROWS)])

    pl.loop(0, nz)(zchunk)
    plsc.subcore_barrier()

    def g_wait(src_c, k, rs):
        pltpu.make_async_copy(x_hbm.at[src_c.at[k]], rows[rs], semG[rs]).wait()

    def g_start(src_c, k, rs):
        pltpu.async_copy(x_hbm.at[src_c.at[k]], rows[rs], semG[rs])

    def s_start(dst_c, k, rs):
        pltpu.async_copy(rows[rs], agg_sh.at[dst_c.at[k]], semS[rs], add=True)

    def s_wait(dst_c, k, rs):
        pltpu.make_async_copy(rows[rs], agg_sh.at[dst_c.at[k]], semS[rs]).wait()

    def chunk_start(src_c, dst_c, sem_s, sem_d, j):
        pltpu.async_copy(idx_hbm.at[0, wid, pl.ds(j, CH)], src_c, sem_s)
        pltpu.async_copy(idx_hbm.at[1, wid, pl.ds(j, CH)], dst_c, sem_d)

    def chunk_wait(src_c, dst_c, sem_s, sem_d, j):
        pltpu.make_async_copy(idx_hbm.at[0, wid, pl.ds(j, CH)], src_c,
                              sem_s).wait()
        pltpu.make_async_copy(idx_hbm.at[1, wid, pl.ds(j, CH)], dst_c,
                              sem_d).wait()

    pltpu.sync_copy(idx_hbm.at[0, wid, pl.ds(0, CH)], srcA)
    pltpu.sync_copy(idx_hbm.at[1, wid, pl.ds(0, CH)], dstA)
    pltpu.sync_copy(idx_hbm.at[0, wid, pl.ds(CH, CH)], srcB)
    pltpu.sync_copy(idx_hbm.at[1, wid, pl.ds(CH, CH)], dstB)
    for b in range(NR - 1):
        g_start(srcA, b, b)

    def turn(j, b, last_block):
        src_c, dst_c = (srcA, dstA) if b < CH else (srcB, dstB)
        k, rs = b % CH, b % NR
        pb = (b - 1) % BODY
        pdst = dstA if pb < CH else dstB
        g_wait(src_c, k, rs)
        s_start(dst_c, k, rs)
        if b == 0:
            @pl.when(j >= 1)
            def _drain0():
                s_wait(pdst, pb % CH, pb % NR)
        else:
            s_wait(pdst, pb % CH, pb % NR)
        if not last_block:
            if b == 1:
                @pl.when(j >= 1)
                def _refillB():
                    chunk_start(srcB, dstB, semSB, semDB, j + CH)
            if b == CH:
                chunk_start(srcA, dstA, semSA, semDA, j + BODY)
            if b == 5:
                @pl.when(j >= 1)
                def _waitB():
                    chunk_wait(srcB, dstB, semSB, semDB, j + CH)
            if b == CH + 5:
                chunk_wait(srcA, dstA, semSA, semDA, j + BODY)
            nb = b + NR - 1
            nsrc = srcA if (nb < CH or nb >= BODY) else srcB
            g_start(nsrc, nb % CH, nb % NR)
        else:
            if b + NR - 1 < CH:
                g_start(srcA, b + NR - 1, (b + NR - 1) % NR)

    def body(j):
        for b in range(BODY):
            turn(j, b, last_block=False)

    pl.loop(0, STEADY, step=BODY)(body)
    for b in range(CH):
        turn(STEADY, b, last_block=True)
    s_wait(dstA, CH - 1, (NB - 1) % NR)

    plsc.subcore_barrier()

    @pl.when(sid < NS - 1)
    def _w0():
        pltpu.sync_copy(agg_sh.at[pl.ds(sid * CHUNK, CHUNK)],
                        out_hbm.at[cid, pl.ds(sid * CHUNK, CHUNK)])

    @pl.when(sid == NS - 1)
    def _w1():
        pltpu.sync_copy(agg_sh.at[pl.ds((NS - 1) * CHUNK, LAST_CHUNK)],
                        out_hbm.at[cid, pl.ds((NS - 1) * CHUNK, LAST_CHUNK)])


@functools.partial(
    pl.kernel,
    out_type=jax.ShapeDtypeStruct((NC, N_NODES, D), jnp.float32),
    mesh=plsc.VectorSubcoreMesh(core_axis_name="c", subcore_axis_name="s",
                                num_cores=NC, num_subcores=NS),
    scratch_types=[
        tuple(pltpu.VMEM((CH, EB), jnp.int32) for _ in range(4)),
        tuple(pltpu.VMEM((EB, D), jnp.float32) for _ in range(NR)),
        pltpu.VMEM((ZROWS, D), jnp.float32),
        pltpu.VMEM_SHARED((N_NODES, D), jnp.float32),
        tuple(pltpu.SemaphoreType.DMA for _ in range(4)),
        tuple(pltpu.SemaphoreType.DMA for _ in range(NR)),
        tuple(pltpu.SemaphoreType.DMA for _ in range(NR)),
    ],
)
def _sc_aggregate(x_hbm, idx_hbm, out_hbm, idxs, rows, zbuf, agg_sh,
                  semI, semG, semS):
    _sc_body(x_hbm, idx_hbm, out_hbm, idxs, rows, zbuf, agg_sh,
             semI, semG, semS)


ROW_BLK = 5000
GRID = N_NODES // ROW_BLK


def _tc_body(x_ref, p_ref, ws_ref, wn_ref, bg_ref, w1_ref, b1_ref,
             w2_ref, b2_ref, out_ref, acc_ref):
    i = pl.program_id(0)

    @pl.when(i == 0)
    def _init():
        acc_ref[...] = jnp.zeros_like(acc_ref)

    xb = x_ref[...]
    ab = p_ref[0] + p_ref[1]
    h = (jnp.dot(xb, ws_ref[...], preferred_element_type=jnp.float32)
         + jnp.dot(ab, wn_ref[...], preferred_element_type=jnp.float32)
         + bg_ref[...])
    h = jnp.maximum(h, 0.0)
    acc_ref[...] += jnp.sum(h, axis=0, keepdims=True)

    @pl.when(i == GRID - 1)
    def _final():
        emb = acc_ref[...] * (1.0 / N_NODES)
        z = jnp.maximum(
            jnp.dot(emb, w1_ref[...], preferred_element_type=jnp.float32)
            + b1_ref[...], 0.0)
        out_ref[...] = (jnp.dot(z, w2_ref[...], preferred_element_type=jnp.float32)
                        + b2_ref[...])


def _tc_finish(x, partials, W_self, W_nbr, b_gnn, W1, b1, W2, b2):
    return pl.pallas_call(
        _tc_body,
        grid=(GRID,),
        in_specs=[
            pl.BlockSpec((ROW_BLK, D), lambda i: (i, 0)),
            pl.BlockSpec((NC, ROW_BLK, D), lambda i: (0, i, 0)),
            pl.BlockSpec((D, D), lambda i: (0, 0)),
            pl.BlockSpec((D, D), lambda i: (0, 0)),
            pl.BlockSpec((1, D), lambda i: (0, 0)),
            pl.BlockSpec((D, D), lambda i: (0, 0)),
            pl.BlockSpec((1, D), lambda i: (0, 0)),
            pl.BlockSpec((D, NUM_CLASSES), lambda i: (0, 0)),
            pl.BlockSpec((1, NUM_CLASSES), lambda i: (0, 0)),
        ],
        out_specs=pl.BlockSpec((1, NUM_CLASSES), lambda i: (0, 0)),
        out_shape=jax.ShapeDtypeStruct((1, NUM_CLASSES), jnp.float32),
        scratch_shapes=[pltpu.VMEM((1, D), jnp.float32)],
    )(x, partials, W_self, W_nbr, b_gnn, W1, b1, W2, b2)


RELAY_GRID = 4


def _relayout_body(in_ref, out_ref):
    out_ref[...] = in_ref[...].reshape(2, NW // RELAY_GRID, NB, EB)


def _relayout(ei):
    return pl.pallas_call(
        _relayout_body,
        grid=(RELAY_GRID,),
        in_specs=[pl.BlockSpec((2, N_EDGES // RELAY_GRID), lambda g: (0, g))],
        out_specs=pl.BlockSpec((2, NW // RELAY_GRID, NB, EB),
                               lambda g: (0, g, 0, 0)),
        out_shape=jax.ShapeDtypeStruct((2, NW, NB, EB), jnp.int32),
    )(ei)


def kernel(x, edge_index, W_self, W_nbr, b_gnn, W1, b1, W2, b2):
    idx = _relayout(edge_index.astype(jnp.int32))
    partials = _sc_aggregate(x, idx)
    return _tc_finish(x, partials,
                      W_self, W_nbr, b_gnn.reshape(1, D),
                      W1, b1.reshape(1, D), W2, b2.reshape(1, NUM_CLASSES))
